# baseline (speedup 1.0000x reference)
import functools

import jax
import jax.numpy as jnp
from jax import lax
from jax.experimental import pallas as pl
from jax.experimental.pallas import tpu as pltpu
from jax.experimental.pallas import tpu_sc as plsc

_E = 16
_H = 1024
_I = 512
_NG = 4
_GSZ = _E // _NG
_RSF = 2.5
_EPS = 1e-20
_T = 2048
_NB = 256
_BPG = _T // _NB
_NS = _NG * _T
_NW = 32
_TPW = _T // _NW
_CH = 512
_CCH = 16


def _sig(v):
    return jax.nn.sigmoid(v)


def _dispatch_body(x_ref, rwt_ref, sg_ref, su_ref, sd_ref,
                   inv0_ref, inv1_ref, w0_ref, w1_ref, counts_ref, ysh_ref,
                   gmask_s, w16_s, c4_s):
    s = pl.program_id(0)
    T = _T

    @pl.when(s == 0)
    def _router():
        x = x_ref[...]
        logits = jnp.dot(x, rwt_ref[...], preferred_element_type=jnp.float32)
        scores = _sig(logits)
        gsums = []
        for g in range(_NG):
            a = scores[:, 4 * g + 0:4 * g + 1]
            b = scores[:, 4 * g + 1:4 * g + 2]
            c = scores[:, 4 * g + 2:4 * g + 3]
            d = scores[:, 4 * g + 3:4 * g + 4]
            s1 = jnp.maximum(a, b); s2 = jnp.minimum(a, b)
            s3 = jnp.maximum(c, d); s4 = jnp.minimum(c, d)
            m = jnp.maximum(s1, s3)
            sec = jnp.maximum(jnp.minimum(s1, s3), jnp.maximum(s2, s4))
            gsums.append(m + sec)
        gs = jnp.concatenate(gsums, axis=1)
        cidx = jax.lax.broadcasted_iota(jnp.int32, (T, _NG), 1)
        m1 = jnp.max(gs, axis=1, keepdims=True)
        i1 = jnp.min(jnp.where(gs == m1, cidx, 9), axis=1, keepdims=True)
        e1 = cidx == i1
        gs2 = jnp.where(e1, -jnp.inf, gs)
        m2 = jnp.max(gs2, axis=1, keepdims=True)
        i2 = jnp.min(jnp.where(gs2 == m2, cidx, 9), axis=1, keepdims=True)
        gmask = jnp.logical_or(e1, cidx == i2).astype(jnp.float32)
        emask = jnp.concatenate(
            [jnp.broadcast_to(gmask[:, g:g + 1], (T, _GSZ))
             for g in range(_NG)], axis=1)
        masked = scores * emask
        denom = jnp.sum(masked, axis=1, keepdims=True)
        gmask_s[...] = gmask
        w16_s[...] = masked / (denom + _EPS) * _RSF

    @pl.when(s > 0)
    def _shared():
        xb = x_ref[pl.ds((s - 1) * _NB, _NB), :]
        gsh = jnp.dot(xb, sg_ref[...], preferred_element_type=jnp.float32)
        ush = jnp.dot(xb, su_ref[...], preferred_element_type=jnp.float32)
        ysh_ref[...] = jnp.dot(gsh * _sig(gsh) * ush, sd_ref[...],
                               preferred_element_type=jnp.float32)

    for rb in range(T // _CH):
        @pl.when(s == rb + 1)
        def _tri(rb=rb):
            ri = jax.lax.broadcasted_iota(jnp.int32, (_CH, T), 0) + rb * _CH
            ci = jax.lax.broadcasted_iota(jnp.int32, (_CH, T), 1)
            trilc = (ci <= ri).astype(jnp.float32)
            c4_s[pl.ds(rb * _CH, _CH), :] = jnp.dot(
                trilc, gmask_s[...], preferred_element_type=jnp.float32)

    @pl.when(s == _BPG)
    def _finalize():
        c4 = c4_s[...]
        gmask = gmask_s[...]
        w16 = w16_s[...]
        counts_ref[...] = c4[T - 1:T, :].astype(jnp.int32)
        cidx = jax.lax.broadcasted_iota(jnp.int32, (T, _NG), 1)
        flat4 = cidx.astype(jnp.float32) * T + c4 - 1.0
        selb = gmask > 0
        inv0f = jnp.min(jnp.where(selb, flat4, 1e9), axis=1, keepdims=True)
        inv1f = jnp.max(jnp.where(selb, flat4, -1.0), axis=1, keepdims=True)
        inv0_ref[...] = inv0f.astype(jnp.int32)
        inv1_ref[...] = inv1f.astype(jnp.int32)
        g0 = jnp.floor(inv0f / T)
        g1 = jnp.floor(inv1f / T)
        w0 = jnp.zeros((T, _GSZ), jnp.float32)
        w1 = jnp.zeros((T, _GSZ), jnp.float32)
        for g in range(_NG):
            wg = w16[:, 4 * g:4 * g + 4]
            w0 = w0 + jnp.where(g0 == g, wg, 0.0)
            w1 = w1 + jnp.where(g1 == g, wg, 0.0)
        pad = jnp.zeros((T, 128 - _GSZ), jnp.float32)
        w0_ref[...] = jnp.concatenate([w0, pad], axis=1)
        w1_ref[...] = jnp.concatenate([w1, pad], axis=1)


def _dispatch(x, rwt, sg, su, sd):
    return pl.pallas_call(
        _dispatch_body,
        grid=(1 + _BPG,),
        in_specs=[
            pl.BlockSpec((_T, _H), lambda s: (0, 0)),
            pl.BlockSpec((_H, _E), lambda s: (0, 0)),
            pl.BlockSpec((_H, _I), lambda s: (0, 0)),
            pl.BlockSpec((_H, _I), lambda s: (0, 0)),
            pl.BlockSpec((_I, _H), lambda s: (0, 0)),
        ],
        out_specs=(
            pl.BlockSpec((_T, 1), lambda s: (0, 0)),
            pl.BlockSpec((_T, 1), lambda s: (0, 0)),
            pl.BlockSpec((_T, 128), lambda s: (0, 0)),
            pl.BlockSpec((_T, 128), lambda s: (0, 0)),
            pl.BlockSpec((1, _NG), lambda s: (0, 0)),
            pl.BlockSpec((_NB, _H), lambda s: (jnp.maximum(s - 1, 0), 0)),
        ),
        out_shape=(
            jax.ShapeDtypeStruct((_T, 1), jnp.int32),
            jax.ShapeDtypeStruct((_T, 1), jnp.int32),
            jax.ShapeDtypeStruct((_T, 128), jnp.float32),
            jax.ShapeDtypeStruct((_T, 128), jnp.float32),
            jax.ShapeDtypeStruct((1, _NG), jnp.int32),
            jax.ShapeDtypeStruct((_T, _H), jnp.float32),
        ),
        scratch_shapes=[
            pltpu.VMEM((_T, _NG), jnp.float32),
            pltpu.VMEM((_T, _E), jnp.float32),
            pltpu.VMEM((_T, _NG), jnp.float32),
        ],
        compiler_params=pltpu.CompilerParams(
            dimension_semantics=("arbitrary",)),
    )(x, rwt, sg, su, sd)


def _sc_scatter(x, w0, w1, inv0, inv1):
    mesh = plsc.VectorSubcoreMesh(core_axis_name="c", subcore_axis_name="s")

    @functools.partial(
        pl.kernel, mesh=mesh,
        out_type=(jax.ShapeDtypeStruct((_NS, _H), jnp.float32),
                  jax.ShapeDtypeStruct((_NS, 128), jnp.float32)),
        scratch_types=[
            pltpu.VMEM((_TPW,), jnp.int32),
            pltpu.VMEM((_TPW,), jnp.int32),
            pltpu.VMEM((_TPW, _H), jnp.float32),
            pltpu.VMEM((_TPW, 128), jnp.float32),
            pltpu.VMEM((_TPW, 128), jnp.float32),
            pltpu.SemaphoreType.DMA,
        ],
    )
    def k(x_hbm, w0_hbm, w1_hbm, i0_hbm, i1_hbm, xs_hbm, ws_hbm,
          i0_v, i1_v, x_v, w0_v, w1_v, sem):
        wid = lax.axis_index("s") * 2 + lax.axis_index("c")
        base = wid * _TPW
        pltpu.sync_copy(i0_hbm.at[pl.ds(base, _TPW)], i0_v)
        pltpu.sync_copy(i1_hbm.at[pl.ds(base, _TPW)], i1_v)
        pltpu.sync_copy(x_hbm.at[pl.ds(base, _TPW)], x_v)
        pltpu.sync_copy(w0_hbm.at[pl.ds(base, _TPW)], w0_v)
        pltpu.sync_copy(w1_hbm.at[pl.ds(base, _TPW)], w1_v)
        cps = [pltpu.async_copy(x_v, xs_hbm.at[i0_v], sem),
               pltpu.async_copy(x_v, xs_hbm.at[i1_v], sem),
               pltpu.async_copy(w0_v, ws_hbm.at[i0_v], sem),
               pltpu.async_copy(w1_v, ws_hbm.at[i1_v], sem)]
        for cp in cps:
            cp.wait()

    return k(x, w0, w1, inv0, inv1)


def _routed_body(counts_ref, xs_ref, ws_ref, gate_ref, up_ref, down_ref,
                 ys_ref):
    g = pl.program_id(0)
    b = pl.program_id(1)

    @pl.when(b * _NB < counts_ref[0, g])
    def _():
        xb = xs_ref[...]
        w4 = ws_ref[:, 0:_GSZ]
        acc = jnp.zeros((_NB, _H), jnp.float32)
        for e in range(_GSZ):
            ge = jnp.dot(xb, gate_ref[e], preferred_element_type=jnp.float32)
            ue = jnp.dot(xb, up_ref[e], preferred_element_type=jnp.float32)
            he = (ue * w4[:, e:e + 1]) * (ge * _sig(ge))
            acc = acc + jnp.dot(he, down_ref[e],
                                preferred_element_type=jnp.float32)
        ys_ref[...] = acc


def _routed(counts, xs, ws, gate_w, up_w, down_w):
    return pl.pallas_call(
        _routed_body,
        grid=(_NG, _BPG),
        in_specs=[
            pl.BlockSpec(memory_space=pltpu.SMEM),
            pl.BlockSpec((_NB, _H), lambda g, b: (g * _BPG + b, 0)),
            pl.BlockSpec((_NB, 128), lambda g, b: (g * _BPG + b, 0)),
            pl.BlockSpec((_GSZ, _H, _I), lambda g, b: (g, 0, 0)),
            pl.BlockSpec((_GSZ, _H, _I), lambda g, b: (g, 0, 0)),
            pl.BlockSpec((_GSZ, _I, _H), lambda g, b: (g, 0, 0)),
        ],
        out_specs=pl.BlockSpec((_NB, _H), lambda g, b: (g * _BPG + b, 0)),
        out_shape=jax.ShapeDtypeStruct((_NS, _H), jnp.float32),
        compiler_params=pltpu.CompilerParams(
            dimension_semantics=("arbitrary", "arbitrary")),
    )(counts, xs, ws, gate_w, up_w, down_w)


def _sc_combine(ys, ysh, inv0, inv1):
    mesh = plsc.VectorSubcoreMesh(core_axis_name="c", subcore_axis_name="s")
    NCH = _TPW // _CCH

    @functools.partial(
        pl.kernel, mesh=mesh,
        out_type=jax.ShapeDtypeStruct((_T, _H), jnp.float32),
        scratch_types=[
            pltpu.VMEM((_CCH,), jnp.int32),
            pltpu.VMEM((_CCH,), jnp.int32),
            pltpu.VMEM((_CCH, _H), jnp.float32),
            pltpu.VMEM((_CCH, _H), jnp.float32),
            pltpu.VMEM((_CCH, _H), jnp.float32),
            pltpu.SemaphoreType.DMA,
            pltpu.VMEM((_CCH,), jnp.int32),
            pltpu.VMEM((_CCH,), jnp.int32),
            pltpu.VMEM((_CCH, _H), jnp.float32),
            pltpu.VMEM((_CCH, _H), jnp.float32),
            pltpu.VMEM((_CCH, _H), jnp.float32),
            pltpu.SemaphoreType.DMA,
        ],
    )
    def k(ys_hbm, ysh_hbm, i0_hbm, i1_hbm, out_hbm, *bufs):
        wid = lax.axis_index("s") * 2 + lax.axis_index("c")
        ping, pong = bufs[0:6], bufs[6:12]
        sets = (ping, pong)

        def start(j, bs):
            i0_v, i1_v, a_v, b_v, c_v, sem = bs
            base = wid * _TPW + j * _CCH
            pltpu.sync_copy(i0_hbm.at[pl.ds(base, _CCH)], i0_v)
            pltpu.sync_copy(i1_hbm.at[pl.ds(base, _CCH)], i1_v)
            return (pltpu.async_copy(ys_hbm.at[i0_v], a_v, sem),
                    pltpu.async_copy(ys_hbm.at[i1_v], b_v, sem),
                    pltpu.async_copy(ysh_hbm.at[pl.ds(base, _CCH)], c_v, sem))

        cps = [None, None]
        cps[0] = start(0, sets[0])
        for j in range(NCH):
            if j + 1 < NCH:
                cps[(j + 1) % 2] = start(j + 1, sets[(j + 1) % 2])
            i0_v, i1_v, a_v, b_v, c_v, sem = sets[j % 2]
            for cp in cps[j % 2]:
                cp.wait()

            def row(r, carry):
                for k16 in range(_H // 16):
                    sl = pl.ds(k16 * 16, 16)
                    a_v[r, sl] = a_v[r, sl] + b_v[r, sl] + c_v[r, sl]
                return carry

            lax.fori_loop(0, _CCH, row, 0)
            base = wid * _TPW + j * _CCH
            pltpu.sync_copy(a_v, out_hbm.at[pl.ds(base, _CCH)])

    return k(ys, ysh, inv0, inv1)


def kernel(hidden_states, router_w, gate_w, up_w, down_w, shared_gate_w,
           shared_up_w, shared_down_w):
    B, S, Hd = hidden_states.shape
    x = hidden_states.reshape(_T, Hd)
    rwt = router_w.T

    inv0, inv1, w0, w1, counts, ysh = _dispatch(
        x, rwt, shared_gate_w, shared_up_w, shared_down_w)
    inv0 = inv0.reshape(_T)
    inv1 = inv1.reshape(_T)
    xs, ws = _sc_scatter(x, w0, w1, inv0, inv1)
    ys = _routed(counts, xs, ws, gate_w, up_w, down_w)
    out = _sc_combine(ys, ysh, inv0, inv1)
    return out.reshape(B, S, Hd)


# speedup vs baseline: 1.4590x; 1.4590x over previous
import jax
import jax.numpy as jnp
from jax.experimental import pallas as pl
from jax.experimental.pallas import tpu as pltpu

_E = 16
_H = 1024
_I = 512
_NG = 4
_GSZ = _E // _NG
_RSF = 2.5
_EPS = 1e-20
_T = 2048
_NQ = 4
_QR = _T // _NQ


def _sig(v):
    return jax.nn.sigmoid(v)


def _moe_body(x_ref, rwt_ref, gate_ref, up_ref, down_ref, sg_ref, su_ref,
              sd_ref, out_ref, w_ref):
    e = pl.program_id(0)
    T = _T

    @pl.when(e == 0)
    def _router():
        x = x_ref[...]
        logits = jnp.dot(x, rwt_ref[...], preferred_element_type=jnp.float32)
        scores = _sig(logits)
        gsums = []
        for g in range(_NG):
            a = scores[:, 4 * g + 0:4 * g + 1]
            b = scores[:, 4 * g + 1:4 * g + 2]
            c = scores[:, 4 * g + 2:4 * g + 3]
            d = scores[:, 4 * g + 3:4 * g + 4]
            s1 = jnp.maximum(a, b); s2 = jnp.minimum(a, b)
            s3 = jnp.maximum(c, d); s4 = jnp.minimum(c, d)
            m = jnp.maximum(s1, s3)
            sec = jnp.maximum(jnp.minimum(s1, s3), jnp.maximum(s2, s4))
            gsums.append(m + sec)
        gs = jnp.concatenate(gsums, axis=1)
        cidx = jax.lax.broadcasted_iota(jnp.int32, (T, _NG), 1)
        m1 = jnp.max(gs, axis=1, keepdims=True)
        i1 = jnp.min(jnp.where(gs == m1, cidx, 9), axis=1, keepdims=True)
        e1 = cidx == i1
        gs2 = jnp.where(e1, -jnp.inf, gs)
        m2 = jnp.max(gs2, axis=1, keepdims=True)
        i2 = jnp.min(jnp.where(gs2 == m2, cidx, 9), axis=1, keepdims=True)
        gmask = jnp.logical_or(e1, cidx == i2).astype(jnp.float32)
        emask = jnp.concatenate(
            [jnp.broadcast_to(gmask[:, g:g + 1], (T, _GSZ))
             for g in range(_NG)], axis=1)
        masked = scores * emask
        denom = jnp.sum(masked, axis=1, keepdims=True)
        w_ref[...] = masked / (denom + _EPS) * _RSF
        out_ref[...] = jnp.zeros_like(out_ref)

    is_shared = e == _E
    gw = jnp.where(is_shared, sg_ref[...], gate_ref[0])
    uw = jnp.where(is_shared, su_ref[...], up_ref[0])
    dw = jnp.where(is_shared, sd_ref[...], down_ref[0])
    onehot = (jax.lax.broadcasted_iota(jnp.int32, (_E, 1), 0)
              == e).astype(jnp.float32)
    wcol = jnp.where(is_shared, 1.0,
                     jnp.dot(w_ref[...], onehot,
                             preferred_element_type=jnp.float32))
    for q in range(_NQ):
        rows = pl.ds(q * _QR, _QR)
        xq = x_ref[rows, :]
        gq = jnp.dot(xq, gw, preferred_element_type=jnp.float32)
        uq = jnp.dot(xq, uw, preferred_element_type=jnp.float32)
        hq = (uq * wcol[q * _QR:(q + 1) * _QR, :]) * (gq * _sig(gq))
        out_ref[rows, :] += jnp.dot(hq, dw, preferred_element_type=jnp.float32)


def kernel(hidden_states, router_w, gate_w, up_w, down_w, shared_gate_w,
           shared_up_w, shared_down_w):
    B, S, Hd = hidden_states.shape
    x = hidden_states.reshape(_T, Hd)
    rwt = router_w.T

    out = pl.pallas_call(
        _moe_body,
        grid=(_E + 1,),
        in_specs=[
            pl.BlockSpec((_T, _H), lambda e: (0, 0)),
            pl.BlockSpec((_H, _E), lambda e: (0, 0)),
            pl.BlockSpec((1, _H, _I), lambda e: (jnp.minimum(e, _E - 1), 0, 0)),
            pl.BlockSpec((1, _H, _I), lambda e: (jnp.minimum(e, _E - 1), 0, 0)),
            pl.BlockSpec((1, _I, _H), lambda e: (jnp.minimum(e, _E - 1), 0, 0)),
            pl.BlockSpec((_H, _I), lambda e: (0, 0)),
            pl.BlockSpec((_H, _I), lambda e: (0, 0)),
            pl.BlockSpec((_I, _H), lambda e: (0, 0)),
        ],
        out_specs=pl.BlockSpec((_T, _H), lambda e: (0, 0)),
        out_shape=jax.ShapeDtypeStruct((_T, _H), jnp.float32),
        scratch_shapes=[pltpu.VMEM((_T, _E), jnp.float32)],
        compiler_params=pltpu.CompilerParams(
            dimension_semantics=("arbitrary",)),
    )(x, rwt, gate_w, up_w, down_w, shared_gate_w, shared_up_w, shared_down_w)
    return out.reshape(B, S, Hd)
